# baseline (speedup 1.0000x reference)
import functools
import math

import jax
import jax.numpy as jnp
from jax import lax
from jax.experimental import pallas as pl
from jax.experimental.pallas import tpu as pltpu
from jax.experimental.pallas import tpu_sc as plsc

BLK = 128
NC = 2
NS = 16
NW = NC * NS

NEG = -3.4e38


def _gelu(y):
    return 0.5 * y * (1.0 + lax.erf(y * (1.0 / math.sqrt(2.0))))


def _route_meta_body(x_ref, miuT_ref, dest_ref, counts_ref, *, padt, chunk):
    p = pl.program_id(0)
    x = x_ref[0]
    t, _ = x.shape
    m_n = miuT_ref.shape[1]
    s = jnp.dot(x, miuT_ref[...], preferred_element_type=jnp.float32)
    mx = jnp.max(s, axis=1, keepdims=True)
    lane = lax.broadcasted_iota(jnp.int32, s.shape, 1)
    a = jnp.min(jnp.where(s >= mx, lane, m_n), axis=1, keepdims=True)
    oh = (lane == a).astype(jnp.float32)

    counts = jnp.sum(oh, axis=0, keepdims=True)
    countc = jnp.floor((counts + (BLK - 1)) * (1.0 / BLK))
    r8 = lax.broadcasted_iota(jnp.int32, (m_n, m_n), 0)
    c8 = lax.broadcasted_iota(jnp.int32, (m_n, m_n), 1)
    triu_strict = (r8 < c8).astype(jnp.float32)
    starts = jnp.dot(countc, triu_strict, preferred_element_type=jnp.float32) * BLK

    rc = lax.broadcasted_iota(jnp.int32, (chunk, chunk), 0)
    cc = lax.broadcasted_iota(jnp.int32, (chunk, chunk), 1)
    tri = (rc >= cc).astype(jnp.float32)
    base = jnp.zeros((1, m_n), jnp.float32)
    for c in range(t // chunk):
        ohc = oh[c * chunk:(c + 1) * chunk, :]
        ccum = jnp.dot(tri, ohc, preferred_element_type=jnp.float32) + base
        base = base + jnp.sum(ohc, axis=0, keepdims=True)
        dst = (jnp.sum(ohc * starts, axis=1, keepdims=True)
               + jnp.sum(ohc * ccum, axis=1, keepdims=True) - 1.0)
        dest_ref[0, c * chunk:(c + 1) * chunk] = dst.astype(jnp.int32) + p * padt

    counts_ref[0] = counts.astype(jnp.int32)


def _expert_body(widx_ref, vcnt_ref, x_ref, w_ref, b_ref, out_ref):
    g = pl.program_id(0)
    y = jnp.dot(x_ref[...], w_ref[0], preferred_element_type=jnp.float32) + b_ref[0]
    y = _gelu(y)
    valid = lax.broadcasted_iota(jnp.int32, (y.shape[0], 1), 0) < vcnt_ref[g]
    out_ref[...] = jnp.where(valid, y, 0.0)


def _attn_body(bexp_ref, kstart_ref, knum_ref, kcount_ref, vcntq_ref,
               q_ref, ks_ref, out_ref, *, scale, nb, m_n, d):
    b = pl.program_id(0)
    j = pl.program_id(1)
    e = bexp_ref[b * nb + j]
    ks = kstart_ref[b * m_n + e]
    kn = knum_ref[b * m_n + e]
    cnt = kcount_ref[b * m_n + e]
    kn = jnp.where(vcntq_ref[b * nb + j] > 0, kn, 0)
    q = q_ref[...]

    def step(i, carry):
        m, l, acc = carry
        kb = ks_ref[pl.ds((ks + i) * BLK, BLK), :]
        logits = lax.dot_general(
            q, kb, (((1,), (1,)), ((), ())), preferred_element_type=jnp.float32
        ) * scale
        vc = cnt - i * BLK
        mask = lax.broadcasted_iota(jnp.int32, logits.shape, 1) < vc
        att = jnp.where(mask, logits, NEG)
        mnew = jnp.maximum(m, jnp.max(att, axis=1, keepdims=True))
        pexp = jnp.where(mask, jnp.exp(att - mnew), 0.0)
        alpha = jnp.exp(m - mnew)
        lnew = l * alpha + jnp.sum(pexp, axis=1, keepdims=True)
        accnew = acc * alpha + jnp.dot(pexp, kb, preferred_element_type=jnp.float32)
        return (mnew, lnew, accnew)

    m0 = jnp.full((q.shape[0], 1), NEG, jnp.float32)
    l0 = jnp.zeros((q.shape[0], 1), jnp.float32)
    a0 = jnp.zeros((q.shape[0], d), jnp.float32)
    m, l, acc = lax.fori_loop(0, kn, step, (m0, l0, a0))
    out_ref[...] = jnp.where(l > 0, acc / l, 0.0)


def kernel(Q, K, V, miu, W_Q, b_Q, W_K, b_K):
    B, SQ, D = Q.shape
    SK = K.shape[1]
    M = miu.shape[0]
    del V

    PADT = SK + M * BLK
    NB = PADT // BLK
    P = 2 * B

    X4 = jnp.concatenate([Q, K], axis=0)

    dest, counts = pl.pallas_call(
        functools.partial(_route_meta_body, padt=PADT, chunk=512),
        grid=(P,),
        in_specs=[
            pl.BlockSpec((1, SQ, D), lambda p: (p, 0, 0)),
            pl.BlockSpec((D, M), lambda p: (0, 0)),
        ],
        out_specs=[
            pl.BlockSpec((1, SQ, 1), lambda p: (p, 0, 0)),
            pl.BlockSpec((1, 1, M), lambda p: (p, 0, 0)),
        ],
        out_shape=[
            jax.ShapeDtypeStruct((P, SQ, 1), jnp.int32),
            jax.ShapeDtypeStruct((P, 1, M), jnp.int32),
        ],
    )(X4, miu.T)

    dest_flat = dest.reshape(P * SQ)
    counts_i = counts.reshape(P, M)

    countc = (counts_i + BLK - 1) // BLK
    ends = jnp.cumsum(countc, axis=1)
    starts = ends - countc
    jj = jnp.arange(NB, dtype=jnp.int32)[None, None, :]
    inblk = (jj >= starts[:, :, None]) & (jj < ends[:, :, None])
    bexp = jnp.sum(
        inblk * jnp.arange(M, dtype=jnp.int32)[None, :, None], axis=1
    )
    used = jnp.any(inblk, axis=1)
    vcnt = jnp.sum(
        inblk * jnp.clip(
            counts_i[:, :, None] - (jj - starts[:, :, None]) * BLK, 0, BLK),
        axis=1,
    )
    side = (jnp.arange(P, dtype=jnp.int32) >= B).astype(jnp.int32)[:, None]
    w_idx = jnp.where(used, bexp + M * side, 0).astype(jnp.int32).reshape(-1)
    vcnt = jnp.where(used, vcnt, 0).astype(jnp.int32).reshape(-1)
    bexp_q = bexp[:B].astype(jnp.int32).reshape(-1)
    vcnt_q = vcnt[:B * NB]
    kstart = starts[B:].astype(jnp.int32).reshape(-1)
    knum = countc[B:].astype(jnp.int32).reshape(-1)
    kcount = counts_i[B:].astype(jnp.int32).reshape(-1)

    xs = _sc_scatter_rows(X4.reshape(P * SQ, D), dest_flat, P * PADT)

    Ws = jnp.concatenate([W_Q, W_K], axis=0)
    bs = jnp.concatenate([b_Q, b_K], axis=0).reshape(2 * M, 1, D)

    xt = pl.pallas_call(
        _expert_body,
        grid_spec=pltpu.PrefetchScalarGridSpec(
            num_scalar_prefetch=2,
            grid=(P * NB,),
            in_specs=[
                pl.BlockSpec((BLK, D), lambda g, widx, vc: (g, 0)),
                pl.BlockSpec((1, D, D), lambda g, widx, vc: (widx[g], 0, 0)),
                pl.BlockSpec((1, 1, D), lambda g, widx, vc: (widx[g], 0, 0)),
            ],
            out_specs=pl.BlockSpec((BLK, D), lambda g, widx, vc: (g, 0)),
        ),
        out_shape=jax.ShapeDtypeStruct((P * PADT, D), jnp.float32),
    )(w_idx, vcnt, xs, Ws, bs)

    att = pl.pallas_call(
        functools.partial(
            _attn_body, scale=1.0 / math.sqrt(D), nb=NB, m_n=M, d=D),
        grid_spec=pltpu.PrefetchScalarGridSpec(
            num_scalar_prefetch=5,
            grid=(B, NB),
            in_specs=[
                pl.BlockSpec((BLK, D), lambda b, j, *s: (b * NB + j, 0)),
                pl.BlockSpec((PADT, D), lambda b, j, *s: (B + b, 0)),
            ],
            out_specs=pl.BlockSpec((BLK, D), lambda b, j, *s: (b * NB + j, 0)),
        ),
        out_shape=jax.ShapeDtypeStruct((B * PADT, D), jnp.float32),
    )(bexp_q, kstart, knum, kcount, vcnt_q, xt, xt)

    O = _sc_gather_rows(att, dest_flat[:B * SQ])
    return O.reshape(B, SQ, D)


def _sc_mesh():
    return plsc.VectorSubcoreMesh(
        core_axis_name="c", subcore_axis_name="s", num_cores=NC, num_subcores=NS)


def _sc_scatter_rows(x, dest, n_out):
    n, d = x.shape
    rows_per_w = n // NW
    c_sz = min(128, rows_per_w)
    nch = rows_per_w // c_sz

    @functools.partial(
        pl.kernel,
        out_type=jax.ShapeDtypeStruct((n_out, d), jnp.float32),
        mesh=_sc_mesh(),
        scratch_types=[
            pltpu.VMEM((c_sz,), jnp.int32),
            pltpu.VMEM((c_sz, d), jnp.float32),
            pltpu.SemaphoreType.DMA,
        ],
    )
    def _scatter(x_hbm, dest_hbm, out_hbm, idx_v, rows_v, sem):
        wid = lax.axis_index("s") * NC + lax.axis_index("c")
        for c in range(nch):
            base = wid * rows_per_w + c * c_sz
            pltpu.sync_copy(x_hbm.at[pl.ds(base, c_sz)], rows_v)
            pltpu.sync_copy(dest_hbm.at[pl.ds(base, c_sz)], idx_v)
            pltpu.async_copy(rows_v, out_hbm.at[idx_v], sem).wait()

    return _scatter(x, dest)


def _sc_gather_rows(src, idx):
    _, d = src.shape
    n = idx.shape[0]
    rows_per_w = n // NW
    c_sz = min(128, rows_per_w)
    nch = rows_per_w // c_sz

    @functools.partial(
        pl.kernel,
        out_type=jax.ShapeDtypeStruct((n, d), jnp.float32),
        mesh=_sc_mesh(),
        scratch_types=[
            pltpu.VMEM((c_sz,), jnp.int32),
            pltpu.VMEM((c_sz, d), jnp.float32),
            pltpu.SemaphoreType.DMA,
        ],
    )
    def _gather(src_hbm, idx_hbm, out_hbm, idx_v, rows_v, sem):
        wid = lax.axis_index("s") * NC + lax.axis_index("c")
        for c in range(nch):
            base = wid * rows_per_w + c * c_sz
            pltpu.sync_copy(idx_hbm.at[pl.ds(base, c_sz)], idx_v)
            pltpu.async_copy(src_hbm.at[idx_v], rows_v, sem).wait()
            pltpu.sync_copy(rows_v, out_hbm.at[pl.ds(base, c_sz)])

    return _gather(src, idx)
